# baseline (speedup 1.0000x reference)
import functools

import jax
import jax.numpy as jnp
from jax import lax
from jax.experimental import pallas as pl
from jax.experimental.pallas import tpu as pltpu
from jax.experimental.pallas import tpu_sc as plsc

N = 16384
K = 8192
D = 256
BR = 512
BK = 1024
RB = N // BR
KB = K // BK
CCOST = 0.25
CB1 = 2736
CB2 = 5472
BIG = 2**30

NC = 2
NS = 16
NW = NC * NS
BPW = N // NW
CH = 128
NCH = BPW // CH


def _argmin_body(x_ref, e_ref, sx_ref, se_ref, idx_out, mind_out,
                 m0, i0, m1, i1, m2, i2):
    k = pl.program_id(1)

    @pl.when(k == 0)
    def _():
        inf = jnp.full((BR, 1), jnp.inf, jnp.float32)
        zero = jnp.zeros((BR, 1), jnp.int32)
        m0[...] = inf; i0[...] = zero
        m1[...] = inf; i1[...] = zero
        m2[...] = inf; i2[...] = zero

    e_blk = e_ref[pl.ds(k * BK, BK), :]
    mm = lax.dot_general(x_ref[...], e_blk, (((1,), (1,)), ((), ())),
                         preferred_element_type=jnp.float32)
    d = (sx_ref[...] + se_ref[...]) - 2.0 * mm
    col = lax.broadcasted_iota(jnp.int32, (BR, BK), 1) + k * BK

    def update(ms, is_, dm, colm):
        m = jnp.min(dm, axis=1, keepdims=True)
        li = jnp.min(jnp.where(dm == m, colm, BIG), axis=1, keepdims=True)
        better = m < ms[...]
        is_[...] = jnp.where(better, li, is_[...])
        ms[...] = jnp.where(better, m, ms[...])

    @pl.when(k < 2)
    def _():
        update(m0, i0, d, col)

    @pl.when(k == 2)
    def _():
        in0 = col < CB1
        update(m0, i0, jnp.where(in0, d, jnp.inf), jnp.where(in0, col, BIG))
        update(m1, i1, jnp.where(in0, jnp.inf, d), jnp.where(in0, BIG, col))

    @pl.when((k == 3) | (k == 4))
    def _():
        update(m1, i1, d, col)

    @pl.when(k == 5)
    def _():
        in1 = col < CB2
        update(m1, i1, jnp.where(in1, d, jnp.inf), jnp.where(in1, col, BIG))
        update(m2, i2, jnp.where(in1, jnp.inf, d), jnp.where(in1, BIG, col))

    @pl.when(k > 5)
    def _():
        update(m2, i2, d, col)

    @pl.when(k == KB - 1)
    def _():
        def bf16(v):
            return v.astype(jnp.bfloat16).astype(jnp.float32)
        av = bf16(m0[...])
        ai = i0[...]
        ax = m0[...]
        keep1 = av <= m1[...]
        av = bf16(jnp.where(keep1, av, m1[...]))
        ai = jnp.where(keep1, ai, i1[...])
        ax = jnp.where(keep1, ax, m1[...])
        keep2 = av <= m2[...]
        ai = jnp.where(keep2, ai, i2[...])
        ax = jnp.where(keep2, ax, m2[...])
        idx_out[...] = ai
        mind_out[0, 0, 0] = jnp.sum(ax)


def _distance_argmin(flat_x, embeddings, sx, se):
    return pl.pallas_call(
        _argmin_body,
        grid=(RB, KB),
        in_specs=[
            pl.BlockSpec((BR, D), lambda i, k: (i, 0)),
            pl.BlockSpec((K, D), lambda i, k: (0, 0)),
            pl.BlockSpec((BR, 1), lambda i, k: (i, 0)),
            pl.BlockSpec((1, BK), lambda i, k: (0, k)),
        ],
        out_specs=[
            pl.BlockSpec((BR, 1), lambda i, k: (i, 0)),
            pl.BlockSpec((1, 1, 1), lambda i, k: (i, 0, 0),
                         memory_space=pltpu.SMEM),
        ],
        out_shape=[
            jax.ShapeDtypeStruct((N, 1), jnp.int32),
            jax.ShapeDtypeStruct((RB, 1, 1), jnp.float32),
        ],
        scratch_shapes=[
            pltpu.VMEM((BR, 1), jnp.float32),
            pltpu.VMEM((BR, 1), jnp.int32),
            pltpu.VMEM((BR, 1), jnp.float32),
            pltpu.VMEM((BR, 1), jnp.int32),
            pltpu.VMEM((BR, 1), jnp.float32),
            pltpu.VMEM((BR, 1), jnp.int32),
        ],
        compiler_params=pltpu.CompilerParams(
            dimension_semantics=("parallel", "arbitrary"),
        ),
    )(flat_x, embeddings, sx, se)


@functools.partial(
    pl.kernel,
    mesh=plsc.VectorSubcoreMesh(core_axis_name="c", subcore_axis_name="s"),
    out_type=jax.ShapeDtypeStruct((N, D), jnp.float32),
    scratch_types=[
        pltpu.VMEM((NCH, CH), jnp.int32),
        pltpu.VMEM((2, CH, D), jnp.float32),
        pltpu.SemaphoreType.DMA,
        pltpu.SemaphoreType.DMA,
    ],
)
def _sc_gather(table_hbm, idx_hbm, out_hbm, idx_v, rows_v, sem0, sem1):
    wid = lax.axis_index("s") * NC + lax.axis_index("c")
    base = wid * BPW
    pltpu.sync_copy(idx_hbm.at[wid], idx_v)
    sems = [sem0, sem1]
    cur = pltpu.async_copy(table_hbm.at[idx_v.at[0]], rows_v.at[0], sems[0])
    for c in range(NCH):
        nxt = c + 1
        pending = cur
        if nxt < NCH:
            cur = pltpu.async_copy(table_hbm.at[idx_v.at[nxt]],
                                   rows_v.at[nxt % 2], sems[nxt % 2])
        pending.wait()
        pltpu.sync_copy(rows_v.at[c % 2],
                        out_hbm.at[pl.ds(base + c * CH, CH)])


def kernel(x, embeddings):
    flat_x = x.reshape(N, D)
    sx = jnp.sum(flat_x ** 2, axis=1).reshape(N, 1)
    se = jnp.sum(embeddings ** 2, axis=1).reshape(1, K)
    idx2, mind_parts = _distance_argmin(flat_x, embeddings, sx, se)
    encoding_indices = idx2.reshape(N)
    quantized_flat = _sc_gather(embeddings,
                                encoding_indices.reshape(NW, NCH, CH))
    quantized = quantized_flat.reshape(x.shape)
    loss = ((1.0 + CCOST) / (N * D)) * jnp.sum(mind_parts)
    return (quantized, loss, encoding_indices)


# speedup vs baseline: 1.1625x; 1.1625x over previous
import functools

import jax
import jax.numpy as jnp
from jax import lax
from jax.experimental import pallas as pl
from jax.experimental.pallas import tpu as pltpu
from jax.experimental.pallas import tpu_sc as plsc

N = 16384
K = 8192
D = 256
BR = 512
BK = 1024
RB = N // BR
KB = K // BK
CCOST = 0.25
CB1 = 2736
CB2 = 5472
BIGF = 1e30

NC = 2
NS = 16
NW = NC * NS
BPW = N // NW
CH = 128
NCH = BPW // CH


def _argmin_body(x_ref, e_ref, sx_ref, se_ref, idx_out, mind_out,
                 m0, i0, m1, i1, m2, i2):
    k = pl.program_id(1)

    @pl.when(k == 0)
    def _():
        inf = jnp.full((BR, 1), jnp.inf, jnp.float32)
        zero = jnp.zeros((BR, 1), jnp.float32)
        m0[...] = inf; i0[...] = zero
        m1[...] = inf; i1[...] = zero
        m2[...] = inf; i2[...] = zero

    e_blk = e_ref[pl.ds(k * BK, BK), :]
    mm = lax.dot_general(x_ref[...], e_blk, (((1,), (1,)), ((), ())),
                         preferred_element_type=jnp.float32)
    d = (sx_ref[...] + se_ref[...]) - 2.0 * mm
    col = (lax.broadcasted_iota(jnp.int32, (BR, BK), 1).astype(jnp.float32)
           + (k * BK).astype(jnp.float32))

    def update(ms, is_, dm, colm):
        m = jnp.min(dm, axis=1, keepdims=True)
        li = jnp.min(jnp.where(dm == m, colm, BIGF), axis=1, keepdims=True)
        better = m < ms[...]
        is_[...] = jnp.where(better, li, is_[...])
        ms[...] = jnp.where(better, m, ms[...])

    @pl.when(k < 2)
    def _():
        update(m0, i0, d, col)

    @pl.when(k == 2)
    def _():
        in0 = col < float(CB1)
        update(m0, i0, jnp.where(in0, d, jnp.inf), jnp.where(in0, col, BIGF))
        update(m1, i1, jnp.where(in0, jnp.inf, d), jnp.where(in0, BIGF, col))

    @pl.when((k == 3) | (k == 4))
    def _():
        update(m1, i1, d, col)

    @pl.when(k == 5)
    def _():
        in1 = col < float(CB2)
        update(m1, i1, jnp.where(in1, d, jnp.inf), jnp.where(in1, col, BIGF))
        update(m2, i2, jnp.where(in1, jnp.inf, d), jnp.where(in1, BIGF, col))

    @pl.when(k > 5)
    def _():
        update(m2, i2, d, col)

    @pl.when(k == KB - 1)
    def _():
        def bf16(v):
            return v.astype(jnp.bfloat16).astype(jnp.float32)
        av = bf16(m0[...])
        ai = i0[...]
        ax = m0[...]
        keep1 = av <= m1[...]
        av = bf16(jnp.where(keep1, av, m1[...]))
        ai = jnp.where(keep1, ai, i1[...])
        ax = jnp.where(keep1, ax, m1[...])
        keep2 = av <= m2[...]
        ai = jnp.where(keep2, ai, i2[...])
        ax = jnp.where(keep2, ax, m2[...])
        idx_out[...] = ai.astype(jnp.int32)
        mind_out[0, 0, 0] = jnp.sum(ax)


def _distance_argmin(flat_x, embeddings, sx, se):
    return pl.pallas_call(
        _argmin_body,
        grid=(RB, KB),
        in_specs=[
            pl.BlockSpec((BR, D), lambda i, k: (i, 0)),
            pl.BlockSpec((K, D), lambda i, k: (0, 0)),
            pl.BlockSpec((BR, 1), lambda i, k: (i, 0)),
            pl.BlockSpec((1, BK), lambda i, k: (0, k)),
        ],
        out_specs=[
            pl.BlockSpec((BR, 1), lambda i, k: (i, 0)),
            pl.BlockSpec((1, 1, 1), lambda i, k: (i, 0, 0),
                         memory_space=pltpu.SMEM),
        ],
        out_shape=[
            jax.ShapeDtypeStruct((N, 1), jnp.int32),
            jax.ShapeDtypeStruct((RB, 1, 1), jnp.float32),
        ],
        scratch_shapes=[
            pltpu.VMEM((BR, 1), jnp.float32),
            pltpu.VMEM((BR, 1), jnp.float32),
            pltpu.VMEM((BR, 1), jnp.float32),
            pltpu.VMEM((BR, 1), jnp.float32),
            pltpu.VMEM((BR, 1), jnp.float32),
            pltpu.VMEM((BR, 1), jnp.float32),
        ],
        compiler_params=pltpu.CompilerParams(
            dimension_semantics=("parallel", "arbitrary"),
        ),
    )(flat_x, embeddings, sx, se)


@functools.partial(
    pl.kernel,
    mesh=plsc.VectorSubcoreMesh(core_axis_name="c", subcore_axis_name="s"),
    out_type=jax.ShapeDtypeStruct((N, D), jnp.float32),
    scratch_types=[
        pltpu.VMEM((NCH, CH), jnp.int32),
        pltpu.VMEM((2, CH, D), jnp.float32),
        pltpu.SemaphoreType.DMA,
        pltpu.SemaphoreType.DMA,
    ],
)
def _sc_gather(table_hbm, idx_hbm, out_hbm, idx_v, rows_v, sem0, sem1):
    wid = lax.axis_index("s") * NC + lax.axis_index("c")
    base = wid * BPW
    pltpu.sync_copy(idx_hbm.at[wid], idx_v)
    sems = [sem0, sem1]
    cur = pltpu.async_copy(table_hbm.at[idx_v.at[0]], rows_v.at[0], sems[0])
    for c in range(NCH):
        nxt = c + 1
        pending = cur
        if nxt < NCH:
            cur = pltpu.async_copy(table_hbm.at[idx_v.at[nxt]],
                                   rows_v.at[nxt % 2], sems[nxt % 2])
        pending.wait()
        pltpu.sync_copy(rows_v.at[c % 2],
                        out_hbm.at[pl.ds(base + c * CH, CH)])


def kernel(x, embeddings):
    flat_x = x.reshape(N, D)
    sx = jnp.sum(flat_x ** 2, axis=1).reshape(N, 1)
    se = jnp.sum(embeddings ** 2, axis=1).reshape(1, K)
    idx2, mind_parts = _distance_argmin(flat_x, embeddings, sx, se)
    encoding_indices = idx2.reshape(N)
    quantized_flat = _sc_gather(embeddings,
                                encoding_indices.reshape(NW, NCH, CH))
    quantized = quantized_flat.reshape(x.shape)
    loss = ((1.0 + CCOST) / (N * D)) * jnp.sum(mind_parts)
    return (quantized, loss, encoding_indices)
